# Initial kernel scaffold; baseline (speedup 1.0000x reference)
import jax
import jax.numpy as jnp
from jax.experimental import pallas as pl


def kernel(x, edge_index, W1a, b1a, W1b, b1b, g1, be1, W2a, b2a, W2b, b2b, g2, be2, Wf1, bf1, Wf2, bf2):
    raise NotImplementedError("write your pallas kernel here")


# speedup vs baseline: 9.2747x; 9.2747x over previous
import functools

import jax
import jax.numpy as jnp
from jax import lax
from jax.experimental import pallas as pl
from jax.experimental.pallas import tpu as pltpu
from jax.experimental.pallas import tpu_sc as plsc

N = 10000
E = 320000
D_IN = 128
DIM = 32
NUM_CLASSES = 40
BN_EPS = 1e-5

NUM_CORES = 2
NUM_SUBCORES = 16
NUM_WORKERS = NUM_CORES * NUM_SUBCORES

EB = 128
ROWS_TOTAL = 2560
ROWS_PER_TILE = ROWS_TOTAL // NUM_WORKERS
E_PAD = ROWS_TOTAL * EB
ACC_ROWS = N + 112
ZROWS = ACC_ROWS // NUM_SUBCORES

NB = 2000


def _make_segsum():
    mesh = plsc.VectorSubcoreMesh(core_axis_name="c", subcore_axis_name="s")

    @functools.partial(
        pl.kernel,
        out_type=jax.ShapeDtypeStruct((NUM_CORES, ACC_ROWS, DIM), jnp.float32),
        mesh=mesh,
        compiler_params=pltpu.CompilerParams(use_tc_tiling_on_sc=False),
        scratch_types=[
            pltpu.VMEM((ROWS_PER_TILE, EB), jnp.int32),
            pltpu.VMEM((ROWS_PER_TILE, EB), jnp.int32),
            pltpu.VMEM((EB, DIM), jnp.float32),
            pltpu.VMEM((EB, DIM), jnp.float32),
            pltpu.VMEM_SHARED((ACC_ROWS, DIM), jnp.float32),
            pltpu.SemaphoreType.DMA,
            pltpu.SemaphoreType.DMA,
        ],
    )
    def segsum(u_hbm, srcr_hbm, dstr_hbm, zeros_hbm, out_hbm,
               src_v, dst_v, rows_a, rows_b, acc_sh, sem_a, sem_b):
        cid = lax.axis_index("c")
        sid = lax.axis_index("s")
        wid = cid * NUM_SUBCORES + sid
        row_base = wid * ROWS_PER_TILE

        pltpu.sync_copy(srcr_hbm.at[pl.ds(row_base, ROWS_PER_TILE)], src_v)
        pltpu.sync_copy(dstr_hbm.at[pl.ds(row_base, ROWS_PER_TILE)], dst_v)
        pltpu.sync_copy(zeros_hbm.at[pl.ds(sid * ZROWS, ZROWS)],
                        acc_sh.at[pl.ds(sid * ZROWS, ZROWS)])
        plsc.subcore_barrier()

        pltpu.async_copy(u_hbm.at[src_v.at[0]], rows_a, sem_a)

        @pl.loop(0, ROWS_PER_TILE // 2)
        def _(p):
            j = p * 2
            pltpu.make_async_copy(u_hbm.at[src_v.at[j]], rows_a, sem_a).wait()
            pltpu.async_copy(u_hbm.at[src_v.at[j + 1]], rows_b, sem_b)
            pltpu.sync_copy(rows_a, acc_sh.at[dst_v.at[j]], add=True)
            pltpu.make_async_copy(u_hbm.at[src_v.at[j + 1]], rows_b, sem_b).wait()

            @pl.when(j + 2 < ROWS_PER_TILE)
            def _():
                pltpu.async_copy(u_hbm.at[src_v.at[j + 2]], rows_a, sem_a)

            pltpu.sync_copy(rows_b, acc_sh.at[dst_v.at[j + 1]], add=True)

        plsc.subcore_barrier()
        pltpu.sync_copy(acc_sh.at[pl.ds(sid * ZROWS, ZROWS)],
                        out_hbm.at[cid].at[pl.ds(sid * ZROWS, ZROWS)])

    return segsum


_segsum = _make_segsum()


def _proj_body(x_ref, w_ref, o_ref):
    o_ref[...] = jnp.dot(x_ref[...], w_ref[...],
                         preferred_element_type=jnp.float32)


def _proj(x, w):
    return pl.pallas_call(
        _proj_body,
        grid=(N // NB,),
        in_specs=[
            pl.BlockSpec((NB, D_IN), lambda i: (i, 0)),
            pl.BlockSpec((D_IN, DIM), lambda i: (0, 0)),
        ],
        out_specs=pl.BlockSpec((NB, DIM), lambda i: (i, 0)),
        out_shape=jax.ShapeDtypeStruct((N, DIM), jnp.float32),
    )(x, w)


def _mid_body(u_ref, a0_ref, a1_ref, w1b_ref, w2a_ref, s_ref, o_ref):
    b1a = s_ref[0]
    b1b = s_ref[1]
    g1s = s_ref[2]
    be1 = s_ref[3]
    t = jnp.maximum(u_ref[...] + a0_ref[...] + a1_ref[...] + b1a, 0.0)
    h = jnp.dot(t, w1b_ref[...], preferred_element_type=jnp.float32) + b1b
    h = jnp.maximum(h, 0.0)
    h = h * g1s + be1
    o_ref[...] = jnp.dot(h, w2a_ref[...], preferred_element_type=jnp.float32)


def _mid(u, a0, a1, w1b, w2a, scalars):
    return pl.pallas_call(
        _mid_body,
        grid=(N // NB,),
        in_specs=[
            pl.BlockSpec((NB, DIM), lambda i: (i, 0)),
            pl.BlockSpec((NB, DIM), lambda i: (i, 0)),
            pl.BlockSpec((NB, DIM), lambda i: (i, 0)),
            pl.BlockSpec((DIM, DIM), lambda i: (0, 0)),
            pl.BlockSpec((DIM, DIM), lambda i: (0, 0)),
            pl.BlockSpec((4, DIM), lambda i: (0, 0)),
        ],
        out_specs=pl.BlockSpec((NB, DIM), lambda i: (i, 0)),
        out_shape=jax.ShapeDtypeStruct((N, DIM), jnp.float32),
    )(u, a0, a1, w1b, w2a, scalars)


def _final_body(v_ref, a0_ref, a1_ref, w2b_ref, wf1_ref, wf2_ref, s_ref,
                bf2_ref, o_ref):
    b2a = s_ref[0]
    b2b = s_ref[1]
    g2s = s_ref[2]
    be2 = s_ref[3]
    bf1 = s_ref[4]
    t = jnp.maximum(v_ref[...] + a0_ref[...] + a1_ref[...] + b2a, 0.0)
    h = jnp.dot(t, w2b_ref[...], preferred_element_type=jnp.float32) + b2b
    h = h * g2s + be2
    f = jnp.maximum(
        jnp.dot(h, wf1_ref[...], preferred_element_type=jnp.float32) + bf1,
        0.0)
    o = jnp.dot(f, wf2_ref[...], preferred_element_type=jnp.float32)
    o = o + bf2_ref[0]
    m = jnp.max(o, axis=1, keepdims=True)
    lse = m + jnp.log(jnp.sum(jnp.exp(o - m), axis=1, keepdims=True))
    o_ref[...] = o - lse


def _final(v, a0, a1, w2b, wf1, wf2, scalars, bf2):
    return pl.pallas_call(
        _final_body,
        grid=(N // NB,),
        in_specs=[
            pl.BlockSpec((NB, DIM), lambda i: (i, 0)),
            pl.BlockSpec((NB, DIM), lambda i: (i, 0)),
            pl.BlockSpec((NB, DIM), lambda i: (i, 0)),
            pl.BlockSpec((DIM, DIM), lambda i: (0, 0)),
            pl.BlockSpec((DIM, DIM), lambda i: (0, 0)),
            pl.BlockSpec((DIM, NUM_CLASSES), lambda i: (0, 0)),
            pl.BlockSpec((5, DIM), lambda i: (0, 0)),
            pl.BlockSpec((1, NUM_CLASSES), lambda i: (0, 0)),
        ],
        out_specs=pl.BlockSpec((NB, NUM_CLASSES), lambda i: (i, 0)),
        out_shape=jax.ShapeDtypeStruct((N, NUM_CLASSES), jnp.float32),
    )(v, a0, a1, w2b, wf1, wf2, scalars, bf2)


def kernel(x, edge_index, W1a, b1a, W1b, b1b, g1, be1,
           W2a, b2a, W2b, b2b, g2, be2, Wf1, bf1, Wf2, bf2):
    ei = edge_index.astype(jnp.int32)
    src = jnp.concatenate(
        [ei[0], jnp.zeros((E_PAD - E,), jnp.int32)]).reshape(ROWS_TOTAL, EB)
    dst = jnp.concatenate(
        [ei[1], jnp.full((E_PAD - E,), N, jnp.int32)]).reshape(ROWS_TOTAL, EB)
    zeros = jnp.zeros((ACC_ROWS, DIM), jnp.float32)

    inv = 1.0 / jnp.sqrt(1.0 + BN_EPS)
    bcast = lambda b: jnp.broadcast_to(b, (DIM,))
    scal1 = jnp.stack([bcast(b1a), bcast(b1b), bcast(g1) * inv, bcast(be1)])
    scal2 = jnp.stack([bcast(b2a), bcast(b2b), bcast(g2) * inv, bcast(be2),
                       bcast(bf1)])

    u = _proj(x, W1a)
    agg1 = _segsum(u, src, dst, zeros)
    v = _mid(u, agg1[0, :N], agg1[1, :N], W1b, W2a, scal1)
    agg2 = _segsum(v, src, dst, zeros)
    out = _final(v, agg2[0, :N], agg2[1, :N], W2b, Wf1, Wf2, scal2,
                 bf2.reshape(1, NUM_CLASSES))
    return out


# speedup vs baseline: 10.9211x; 1.1775x over previous
import functools

import jax
import jax.numpy as jnp
from jax import lax
from jax.experimental import pallas as pl
from jax.experimental.pallas import tpu as pltpu
from jax.experimental.pallas import tpu_sc as plsc

N = 10000
E = 320000
D_IN = 128
DIM = 32
NUM_CLASSES = 40
BN_EPS = 1e-5

NUM_CORES = 2
NUM_SUBCORES = 16
NUM_WORKERS = NUM_CORES * NUM_SUBCORES

EB = 128
ROWS_TOTAL = 2560
ROWS_PER_TILE = ROWS_TOTAL // NUM_WORKERS
E_PAD = ROWS_TOTAL * EB
ACC_ROWS = N + 112
ZROWS = ACC_ROWS // NUM_SUBCORES

NB = 2000


def _make_segsum():
    mesh = plsc.VectorSubcoreMesh(core_axis_name="c", subcore_axis_name="s")

    nbuf = 8
    nsteps = ROWS_PER_TILE // nbuf

    @functools.partial(
        pl.kernel,
        out_type=jax.ShapeDtypeStruct((NUM_CORES, ACC_ROWS, DIM), jnp.float32),
        mesh=mesh,
        compiler_params=pltpu.CompilerParams(use_tc_tiling_on_sc=False),
        scratch_types=(
            [pltpu.VMEM((ROWS_PER_TILE, EB), jnp.int32)] * 2
            + [pltpu.VMEM((EB, DIM), jnp.float32)] * nbuf
            + [pltpu.VMEM_SHARED((ACC_ROWS, DIM), jnp.float32)]
            + [pltpu.SemaphoreType.DMA] * (2 * nbuf)
        ),
    )
    def segsum(u_hbm, srcr_hbm, dstr_hbm, zeros_hbm, out_hbm,
               src_v, dst_v, *rest):
        rows = rest[:nbuf]
        acc_sh = rest[nbuf]
        gs = rest[nbuf + 1:nbuf + 1 + nbuf]
        ss = rest[nbuf + 1 + nbuf:]
        cid = lax.axis_index("c")
        sid = lax.axis_index("s")
        wid = cid * NUM_SUBCORES + sid
        row_base = wid * ROWS_PER_TILE

        pltpu.sync_copy(srcr_hbm.at[pl.ds(row_base, ROWS_PER_TILE)], src_v)
        pltpu.sync_copy(dstr_hbm.at[pl.ds(row_base, ROWS_PER_TILE)], dst_v)
        pltpu.sync_copy(zeros_hbm.at[pl.ds(sid * ZROWS, ZROWS)],
                        acc_sh.at[pl.ds(sid * ZROWS, ZROWS)])
        plsc.subcore_barrier()

        def gather_start(j, b):
            pltpu.async_copy(u_hbm.at[src_v.at[j]], rows[b], gs[b])

        def gather_wait(j, b):
            pltpu.make_async_copy(u_hbm.at[src_v.at[j]], rows[b], gs[b]).wait()

        def scat_start(j, b):
            pltpu.async_copy(rows[b], acc_sh.at[dst_v.at[j]], ss[b], add=True)

        def scat_wait(j, b):
            pltpu.make_async_copy(rows[b], acc_sh.at[dst_v.at[j]],
                                  ss[b]).wait()

        for b in range(nbuf // 2):
            gather_start(b, b)

        @pl.loop(0, nsteps)
        def _(p):
            j0 = p * nbuf
            for b in range(nbuf):
                j = j0 + b
                gather_wait(j, b)
                scat_start(j, b)
                jn = j + nbuf // 2
                bn = (b + nbuf // 2) % nbuf

                @pl.when(jn < ROWS_PER_TILE)
                def _():
                    @pl.when(jn >= nbuf)
                    def _():
                        scat_wait(jn - nbuf, bn)

                    gather_start(jn, bn)

        for b in range(nbuf):
            last = (nsteps - 1) * nbuf + b
            scat_wait(last, b)

        plsc.subcore_barrier()
        pltpu.sync_copy(acc_sh.at[pl.ds(sid * ZROWS, ZROWS)],
                        out_hbm.at[cid].at[pl.ds(sid * ZROWS, ZROWS)])

    return segsum


_segsum = _make_segsum()


def _proj_body(x_ref, w_ref, o_ref):
    o_ref[...] = jnp.dot(x_ref[...], w_ref[...],
                         preferred_element_type=jnp.float32)


def _proj(x, w):
    return pl.pallas_call(
        _proj_body,
        grid=(N // NB,),
        in_specs=[
            pl.BlockSpec((NB, D_IN), lambda i: (i, 0)),
            pl.BlockSpec((D_IN, DIM), lambda i: (0, 0)),
        ],
        out_specs=pl.BlockSpec((NB, DIM), lambda i: (i, 0)),
        out_shape=jax.ShapeDtypeStruct((N, DIM), jnp.float32),
    )(x, w)


def _mid_body(u_ref, a0_ref, a1_ref, w1b_ref, w2a_ref, s_ref, o_ref):
    b1a = s_ref[0]
    b1b = s_ref[1]
    g1s = s_ref[2]
    be1 = s_ref[3]
    t = jnp.maximum(u_ref[...] + a0_ref[...] + a1_ref[...] + b1a, 0.0)
    h = jnp.dot(t, w1b_ref[...], preferred_element_type=jnp.float32) + b1b
    h = jnp.maximum(h, 0.0)
    h = h * g1s + be1
    o_ref[...] = jnp.dot(h, w2a_ref[...], preferred_element_type=jnp.float32)


def _mid(u, a0, a1, w1b, w2a, scalars):
    return pl.pallas_call(
        _mid_body,
        grid=(N // NB,),
        in_specs=[
            pl.BlockSpec((NB, DIM), lambda i: (i, 0)),
            pl.BlockSpec((NB, DIM), lambda i: (i, 0)),
            pl.BlockSpec((NB, DIM), lambda i: (i, 0)),
            pl.BlockSpec((DIM, DIM), lambda i: (0, 0)),
            pl.BlockSpec((DIM, DIM), lambda i: (0, 0)),
            pl.BlockSpec((4, DIM), lambda i: (0, 0)),
        ],
        out_specs=pl.BlockSpec((NB, DIM), lambda i: (i, 0)),
        out_shape=jax.ShapeDtypeStruct((N, DIM), jnp.float32),
    )(u, a0, a1, w1b, w2a, scalars)


def _final_body(v_ref, a0_ref, a1_ref, w2b_ref, wf1_ref, wf2_ref, s_ref,
                bf2_ref, o_ref):
    b2a = s_ref[0]
    b2b = s_ref[1]
    g2s = s_ref[2]
    be2 = s_ref[3]
    bf1 = s_ref[4]
    t = jnp.maximum(v_ref[...] + a0_ref[...] + a1_ref[...] + b2a, 0.0)
    h = jnp.dot(t, w2b_ref[...], preferred_element_type=jnp.float32) + b2b
    h = h * g2s + be2
    f = jnp.maximum(
        jnp.dot(h, wf1_ref[...], preferred_element_type=jnp.float32) + bf1,
        0.0)
    o = jnp.dot(f, wf2_ref[...], preferred_element_type=jnp.float32)
    o = o + bf2_ref[0]
    m = jnp.max(o, axis=1, keepdims=True)
    lse = m + jnp.log(jnp.sum(jnp.exp(o - m), axis=1, keepdims=True))
    o_ref[...] = o - lse


def _final(v, a0, a1, w2b, wf1, wf2, scalars, bf2):
    return pl.pallas_call(
        _final_body,
        grid=(N // NB,),
        in_specs=[
            pl.BlockSpec((NB, DIM), lambda i: (i, 0)),
            pl.BlockSpec((NB, DIM), lambda i: (i, 0)),
            pl.BlockSpec((NB, DIM), lambda i: (i, 0)),
            pl.BlockSpec((DIM, DIM), lambda i: (0, 0)),
            pl.BlockSpec((DIM, DIM), lambda i: (0, 0)),
            pl.BlockSpec((DIM, NUM_CLASSES), lambda i: (0, 0)),
            pl.BlockSpec((5, DIM), lambda i: (0, 0)),
            pl.BlockSpec((1, NUM_CLASSES), lambda i: (0, 0)),
        ],
        out_specs=pl.BlockSpec((NB, NUM_CLASSES), lambda i: (i, 0)),
        out_shape=jax.ShapeDtypeStruct((N, NUM_CLASSES), jnp.float32),
    )(v, a0, a1, w2b, wf1, wf2, scalars, bf2)


def kernel(x, edge_index, W1a, b1a, W1b, b1b, g1, be1,
           W2a, b2a, W2b, b2b, g2, be2, Wf1, bf1, Wf2, bf2):
    ei = edge_index.astype(jnp.int32)
    src = jnp.concatenate(
        [ei[0], jnp.zeros((E_PAD - E,), jnp.int32)]).reshape(ROWS_TOTAL, EB)
    dst = jnp.concatenate(
        [ei[1], jnp.full((E_PAD - E,), N, jnp.int32)]).reshape(ROWS_TOTAL, EB)
    zeros = jnp.zeros((ACC_ROWS, DIM), jnp.float32)

    inv = 1.0 / jnp.sqrt(1.0 + BN_EPS)
    bcast = lambda b: jnp.broadcast_to(b, (DIM,))
    scal1 = jnp.stack([bcast(b1a), bcast(b1b), bcast(g1) * inv, bcast(be1)])
    scal2 = jnp.stack([bcast(b2a), bcast(b2b), bcast(g2) * inv, bcast(be2),
                       bcast(bf1)])

    u = _proj(x, W1a)
    agg1 = _segsum(u, src, dst, zeros)
    v = _mid(u, agg1[0, :N], agg1[1, :N], W1b, W2a, scal1)
    agg2 = _segsum(v, src, dst, zeros)
    out = _final(v, agg2[0, :N], agg2[1, :N], W2b, Wf1, Wf2, scal2,
                 bf2.reshape(1, NUM_CLASSES))
    return out


# speedup vs baseline: 10.9676x; 1.0043x over previous
import functools

import jax
import jax.numpy as jnp
from jax import lax
from jax.experimental import pallas as pl
from jax.experimental.pallas import tpu as pltpu
from jax.experimental.pallas import tpu_sc as plsc

N = 10000
E = 320000
D_IN = 128
DIM = 32
NUM_CLASSES = 40
BN_EPS = 1e-5

NUM_CORES = 2
NUM_SUBCORES = 16
NUM_WORKERS = NUM_CORES * NUM_SUBCORES

EB = 128
EB2 = 256
ROWS_TOTAL = 2560
ROWS_PER_TILE = ROWS_TOTAL // NUM_WORKERS
E_PAD = ROWS_TOTAL * EB
ACC_ROWS = N + 112
ZROWS = ACC_ROWS // NUM_SUBCORES

NB = 2000


def _make_segsum():
    mesh = plsc.VectorSubcoreMesh(core_axis_name="c", subcore_axis_name="s")

    nbat = ROWS_PER_TILE // (EB2 // EB)
    nbuf = 8
    nsteps = nbat // nbuf

    @functools.partial(
        pl.kernel,
        out_type=jax.ShapeDtypeStruct((NUM_CORES, ACC_ROWS, DIM), jnp.float32),
        mesh=mesh,
        compiler_params=pltpu.CompilerParams(use_tc_tiling_on_sc=False),
        scratch_types=(
            [pltpu.VMEM((nbat, EB2), jnp.int32)] * 2
            + [pltpu.VMEM((EB2, DIM), jnp.float32)] * nbuf
            + [pltpu.VMEM_SHARED((ACC_ROWS, DIM), jnp.float32)]
            + [pltpu.SemaphoreType.DMA] * (2 * nbuf)
        ),
    )
    def segsum(u_hbm, srcr_hbm, dstr_hbm, zeros_hbm, out_hbm,
               src_v, dst_v, *rest):
        rows = rest[:nbuf]
        acc_sh = rest[nbuf]
        gs = rest[nbuf + 1:nbuf + 1 + nbuf]
        ss = rest[nbuf + 1 + nbuf:]
        cid = lax.axis_index("c")
        sid = lax.axis_index("s")
        wid = cid * NUM_SUBCORES + sid
        row_base = wid * nbat

        pltpu.sync_copy(srcr_hbm.at[pl.ds(row_base, nbat)], src_v)
        pltpu.sync_copy(dstr_hbm.at[pl.ds(row_base, nbat)], dst_v)
        pltpu.sync_copy(zeros_hbm.at[pl.ds(sid * ZROWS, ZROWS)],
                        acc_sh.at[pl.ds(sid * ZROWS, ZROWS)])
        plsc.subcore_barrier()

        def gather_start(j, b):
            pltpu.async_copy(u_hbm.at[src_v.at[j]], rows[b], gs[b])

        def gather_wait(j, b):
            pltpu.make_async_copy(u_hbm.at[src_v.at[j]], rows[b], gs[b]).wait()

        def scat_start(j, b):
            pltpu.async_copy(rows[b], acc_sh.at[dst_v.at[j]], ss[b], add=True)

        def scat_wait(j, b):
            pltpu.make_async_copy(rows[b], acc_sh.at[dst_v.at[j]],
                                  ss[b]).wait()

        for b in range(nbuf // 2):
            gather_start(b, b)

        @pl.loop(0, nsteps)
        def _(p):
            j0 = p * nbuf
            for b in range(nbuf):
                j = j0 + b
                gather_wait(j, b)
                scat_start(j, b)
                jn = j + nbuf // 2
                bn = (b + nbuf // 2) % nbuf

                @pl.when(jn < nbat)
                def _():
                    @pl.when(jn >= nbuf)
                    def _():
                        scat_wait(jn - nbuf, bn)

                    gather_start(jn, bn)

        for b in range(nbuf):
            last = (nsteps - 1) * nbuf + b
            scat_wait(last, b)

        plsc.subcore_barrier()
        pltpu.sync_copy(acc_sh.at[pl.ds(sid * ZROWS, ZROWS)],
                        out_hbm.at[cid].at[pl.ds(sid * ZROWS, ZROWS)])

    return segsum


_segsum = _make_segsum()


def _proj_body(x_ref, w_ref, o_ref):
    o_ref[...] = jnp.dot(x_ref[...], w_ref[...],
                         preferred_element_type=jnp.float32)


def _proj(x, w):
    return pl.pallas_call(
        _proj_body,
        grid=(N // NB,),
        in_specs=[
            pl.BlockSpec((NB, D_IN), lambda i: (i, 0)),
            pl.BlockSpec((D_IN, DIM), lambda i: (0, 0)),
        ],
        out_specs=pl.BlockSpec((NB, DIM), lambda i: (i, 0)),
        out_shape=jax.ShapeDtypeStruct((N, DIM), jnp.float32),
    )(x, w)


def _mid_body(u_ref, a0_ref, a1_ref, w1b_ref, w2a_ref, s_ref, o_ref):
    b1a = s_ref[0]
    b1b = s_ref[1]
    g1s = s_ref[2]
    be1 = s_ref[3]
    t = jnp.maximum(u_ref[...] + a0_ref[...] + a1_ref[...] + b1a, 0.0)
    h = jnp.dot(t, w1b_ref[...], preferred_element_type=jnp.float32) + b1b
    h = jnp.maximum(h, 0.0)
    h = h * g1s + be1
    o_ref[...] = jnp.dot(h, w2a_ref[...], preferred_element_type=jnp.float32)


def _mid(u, a0, a1, w1b, w2a, scalars):
    return pl.pallas_call(
        _mid_body,
        grid=(N // NB,),
        in_specs=[
            pl.BlockSpec((NB, DIM), lambda i: (i, 0)),
            pl.BlockSpec((NB, DIM), lambda i: (i, 0)),
            pl.BlockSpec((NB, DIM), lambda i: (i, 0)),
            pl.BlockSpec((DIM, DIM), lambda i: (0, 0)),
            pl.BlockSpec((DIM, DIM), lambda i: (0, 0)),
            pl.BlockSpec((4, DIM), lambda i: (0, 0)),
        ],
        out_specs=pl.BlockSpec((NB, DIM), lambda i: (i, 0)),
        out_shape=jax.ShapeDtypeStruct((N, DIM), jnp.float32),
    )(u, a0, a1, w1b, w2a, scalars)


def _final_body(v_ref, a0_ref, a1_ref, w2b_ref, wf1_ref, wf2_ref, s_ref,
                bf2_ref, o_ref):
    b2a = s_ref[0]
    b2b = s_ref[1]
    g2s = s_ref[2]
    be2 = s_ref[3]
    bf1 = s_ref[4]
    t = jnp.maximum(v_ref[...] + a0_ref[...] + a1_ref[...] + b2a, 0.0)
    h = jnp.dot(t, w2b_ref[...], preferred_element_type=jnp.float32) + b2b
    h = h * g2s + be2
    f = jnp.maximum(
        jnp.dot(h, wf1_ref[...], preferred_element_type=jnp.float32) + bf1,
        0.0)
    o = jnp.dot(f, wf2_ref[...], preferred_element_type=jnp.float32)
    o = o + bf2_ref[0]
    m = jnp.max(o, axis=1, keepdims=True)
    lse = m + jnp.log(jnp.sum(jnp.exp(o - m), axis=1, keepdims=True))
    o_ref[...] = o - lse


def _final(v, a0, a1, w2b, wf1, wf2, scalars, bf2):
    return pl.pallas_call(
        _final_body,
        grid=(N // NB,),
        in_specs=[
            pl.BlockSpec((NB, DIM), lambda i: (i, 0)),
            pl.BlockSpec((NB, DIM), lambda i: (i, 0)),
            pl.BlockSpec((NB, DIM), lambda i: (i, 0)),
            pl.BlockSpec((DIM, DIM), lambda i: (0, 0)),
            pl.BlockSpec((DIM, DIM), lambda i: (0, 0)),
            pl.BlockSpec((DIM, NUM_CLASSES), lambda i: (0, 0)),
            pl.BlockSpec((5, DIM), lambda i: (0, 0)),
            pl.BlockSpec((1, NUM_CLASSES), lambda i: (0, 0)),
        ],
        out_specs=pl.BlockSpec((NB, NUM_CLASSES), lambda i: (i, 0)),
        out_shape=jax.ShapeDtypeStruct((N, NUM_CLASSES), jnp.float32),
    )(v, a0, a1, w2b, wf1, wf2, scalars, bf2)


def kernel(x, edge_index, W1a, b1a, W1b, b1b, g1, be1,
           W2a, b2a, W2b, b2b, g2, be2, Wf1, bf1, Wf2, bf2):
    ei = edge_index.astype(jnp.int32)
    src = jnp.concatenate(
        [ei[0], jnp.zeros((E_PAD - E,), jnp.int32)]
    ).reshape(E_PAD // EB2, EB2)
    dst = jnp.concatenate(
        [ei[1], jnp.full((E_PAD - E,), N, jnp.int32)]
    ).reshape(E_PAD // EB2, EB2)
    zeros = jnp.zeros((ACC_ROWS, DIM), jnp.float32)

    inv = 1.0 / jnp.sqrt(1.0 + BN_EPS)
    bcast = lambda b: jnp.broadcast_to(b, (DIM,))
    scal1 = jnp.stack([bcast(b1a), bcast(b1b), bcast(g1) * inv, bcast(be1)])
    scal2 = jnp.stack([bcast(b2a), bcast(b2b), bcast(g2) * inv, bcast(be2),
                       bcast(bf1)])

    u = _proj(x, W1a)
    agg1 = _segsum(u, src, dst, zeros)
    v = _mid(u, agg1[0, :N], agg1[1, :N], W1b, W2a, scal1)
    agg2 = _segsum(v, src, dst, zeros)
    out = _final(v, agg2[0, :N], agg2[1, :N], W2b, Wf1, Wf2, scal2,
                 bf2.reshape(1, NUM_CLASSES))
    return out
